# Initial kernel scaffold; baseline (speedup 1.0000x reference)
import jax
import jax.numpy as jnp
from jax.experimental import pallas as pl


def kernel(template, search, params):
    raise NotImplementedError("write your pallas kernel here")


# speedup vs baseline: 3.3464x; 3.3464x over previous
import functools

import jax
import jax.numpy as jnp
from jax.experimental import pallas as pl

EPS = 1e-5
BIG = 1e10


def _interp():
    return jax.default_backend() == "cpu"


def _fps_kernel(xT, yT, zT, ox, oy, oz, *, npoint, n):
    x = xT[...]
    y = yT[...]
    z = zT[...]
    b = x.shape[1]
    iota = jax.lax.broadcasted_iota(jnp.int32, (n, b), 0)

    def body(i, carry):
        dists, far = carry
        mask = iota == far
        cx = jnp.sum(jnp.where(mask, x, 0.0), axis=0, keepdims=True)
        cy = jnp.sum(jnp.where(mask, y, 0.0), axis=0, keepdims=True)
        cz = jnp.sum(jnp.where(mask, z, 0.0), axis=0, keepdims=True)
        ox[pl.ds(i, 1), :] = cx
        oy[pl.ds(i, 1), :] = cy
        oz[pl.ds(i, 1), :] = cz
        d = (x - cx) ** 2 + (y - cy) ** 2 + (z - cz) ** 2
        dists = jnp.minimum(dists, d)
        dmax = jnp.max(dists, axis=0, keepdims=True)
        cand = jnp.where(dists == dmax, iota, n)
        far = jnp.min(cand, axis=0, keepdims=True)
        return dists, far

    jax.lax.fori_loop(
        0, npoint, body,
        (jnp.full((n, b), BIG, jnp.float32), jnp.zeros((1, b), jnp.int32)),
    )


def _fps_call(xyz, npoint):
    bsz, n, _ = xyz.shape
    xT = jnp.transpose(xyz[..., 0])
    yT = jnp.transpose(xyz[..., 1])
    zT = jnp.transpose(xyz[..., 2])
    outs = pl.pallas_call(
        functools.partial(_fps_kernel, npoint=npoint, n=n),
        out_shape=[jax.ShapeDtypeStruct((npoint, bsz), jnp.float32)] * 3,
        interpret=_interp(),
    )(xT, yT, zT)
    return jnp.stack([o.T for o in outs], axis=-1)


def _sa_kernel(ctr_ref, ptsT_ref, gsrc_ref, *rest, k, r2, sub, wb_count):
    wbs = rest[:wb_count]
    out_ref = rest[wb_count]
    t = ctr_ref.shape[0]
    n = ptsT_ref.shape[1]
    ctr = ctr_ref[...]
    cx = ctr[:, 0:1]
    cy = ctr[:, 1:2]
    cz = ctr[:, 2:3]
    px = ptsT_ref[0:1, :]
    py = ptsT_ref[1:2, :]
    pz = ptsT_ref[2:3, :]
    d2 = (cx - px) ** 2 + (cy - py) ** 2 + (cz - pz) ** 2
    work0 = jnp.where(d2 < r2, d2, BIG)
    iota = jax.lax.broadcasted_iota(jnp.int32, (t, n), 1)
    gsrc = gsrc_ref[...]
    cin = gsrc.shape[1]
    if wb_count:
        cout = wbs[wb_count - 2].shape[1]
    else:
        cout = cin

    def sel(work):
        m = jnp.min(work, axis=1, keepdims=True)
        cand = jnp.where(work == m, iota, n)
        amin = jnp.min(cand, axis=1, keepdims=True)
        return iota == amin, m < 1e9

    oh0, _ = sel(work0)
    oh0f = oh0.astype(jnp.float32)
    if sub:
        if cin > 3:
            ctrpad = jnp.concatenate(
                [ctr[:, :3], jnp.zeros((t, cin - 3), jnp.float32)], axis=1)
        else:
            ctrpad = ctr[:, :cin]

    def body(_, carry):
        work, mx = carry
        ohi, valid = sel(work)
        work = jnp.where(ohi, BIG, work)
        ohf = jnp.where(valid, ohi.astype(jnp.float32), oh0f)
        g = jnp.dot(ohf, gsrc, preferred_element_type=jnp.float32)
        if sub:
            g = g - ctrpad
        h = g
        for li in range(wb_count // 2):
            w = wbs[2 * li][...]
            b = wbs[2 * li + 1][...]
            h = jnp.dot(h, w, preferred_element_type=jnp.float32) + b
            h = jnp.maximum(h, 0.0)
        return work, jnp.maximum(mx, h)

    _, mx = jax.lax.fori_loop(
        0, k, body, (work0, jnp.full((t, cout), -jnp.inf, jnp.float32)))
    out_ref[...] = mx


def _fold_bn(layers):
    out = []
    for (w, b, g, be, m, v) in layers:
        s = g / jnp.sqrt(v + EPS)
        out.append((w * s[None, :], ((b - m) * s + be)[None, :]))
    return out


def _sa_call(ctr, xyz, gsrc, wbs, k, radius, sub):
    bsz, np_, _ = ctr.shape
    n = xyz.shape[1]
    cin = gsrc.shape[2]
    if wbs:
        cout = wbs[-1][0].shape[1]
    else:
        cout = cin
    tile = min(np_, 128)
    grid = (bsz, np_ // tile)
    ptsT = jnp.transpose(xyz, (0, 2, 1))
    flat_w = [a for wb in wbs for a in wb]
    in_specs = [
        pl.BlockSpec((None, tile, 3), lambda b, t: (b, t, 0)),
        pl.BlockSpec((None, 3, n), lambda b, t: (b, 0, 0)),
        pl.BlockSpec((None, n, cin), lambda b, t: (b, 0, 0)),
    ] + [pl.BlockSpec(a.shape, lambda b, t: (0, 0)) for a in flat_w]
    return pl.pallas_call(
        functools.partial(_sa_kernel, k=k, r2=radius * radius, sub=sub,
                          wb_count=len(flat_w)),
        grid=grid,
        in_specs=in_specs,
        out_specs=pl.BlockSpec((None, tile, cout), lambda b, t: (b, t, 0)),
        out_shape=jax.ShapeDtypeStruct((bsz, np_, cout), jnp.float32),
        interpret=_interp(),
    )(ctr, ptsT, gsrc, *flat_w)


def _head_kernel(x_ref, *rest, nblocks, sigmoid):
    out_ref = rest[-1]
    h = x_ref[...]
    p = 0
    for _ in range(nblocks):
        w = rest[p][...]
        b = rest[p + 1][...]
        s = rest[p + 2][...]
        tt = rest[p + 3][...]
        p += 4
        h = jnp.maximum(jnp.dot(h, w, preferred_element_type=jnp.float32) + b,
                        0.0)
        h = h * s + tt
    wf = rest[p][...]
    bf = rest[p + 1][...]
    o = jnp.dot(h, wf, preferred_element_type=jnp.float32) + bf
    if sigmoid:
        o = jax.nn.sigmoid(o)
    out_ref[...] = o


def _head_call(x, seq, sigmoid=False):
    bsz, np_, cin = x.shape
    flat = []
    for (w, b, g, be, m, v) in seq['blocks']:
        s = g / jnp.sqrt(v + EPS)
        flat += [w, b[None, :], s[:, None], (be - m * s)[:, None]]
    wf, bf = seq['final']
    flat += [wf, bf[None, :]]
    cout = wf.shape[1]
    nblocks = len(seq['blocks'])
    in_specs = [pl.BlockSpec((None, np_, cin), lambda b: (b, 0, 0))] + [
        pl.BlockSpec(a.shape, lambda b: (0, 0)) for a in flat]
    return pl.pallas_call(
        functools.partial(_head_kernel, nblocks=nblocks, sigmoid=sigmoid),
        grid=(bsz,),
        in_specs=in_specs,
        out_specs=pl.BlockSpec((None, np_, cout), lambda b: (b, 0, 0)),
        out_shape=jax.ShapeDtypeStruct((bsz, np_, cout), jnp.float32),
        interpret=_interp(),
    )(x, *flat)


def _voxelize(feat_pm, xyz):
    sg = jnp.array([-5.6, -3.6, -2.4], dtype=jnp.float32)
    vs = jnp.array([0.3, 0.3, 0.3], dtype=jnp.float32)
    dims = (38, 24, 18)
    vcount = dims[0] * dims[1] * dims[2]
    vi = jnp.floor((xyz - sg) / vs).astype(jnp.int32)
    vi = jnp.clip(vi, jnp.zeros(3, jnp.int32), jnp.array(dims, jnp.int32) - 1)
    flat = vi[..., 0] * (dims[1] * dims[2]) + vi[..., 1] * dims[2] + vi[..., 2]

    def one(fb, ib):
        sums = jax.ops.segment_sum(fb, ib, num_segments=vcount)
        cnt = jax.ops.segment_sum(jnp.ones_like(ib, dtype=fb.dtype), ib,
                                  num_segments=vcount)
        mean = sums / jnp.maximum(cnt, 1.0)[:, None]
        c = fb.shape[1]
        return mean.T.reshape(c, dims[0], dims[1], dims[2])

    return jax.vmap(one)(feat_pm, flat)


def _conv3d(x, w, b, stride):
    out = jax.lax.conv_general_dilated(
        x, w, stride, [(1, 1)] * 3, dimension_numbers=('NCDHW', 'OIDHW', 'NCDHW'))
    return out + b[None, :, None, None, None]


def _conv2d(x, w, b):
    out = jax.lax.conv_general_dilated(
        x, w, (1, 1), [(1, 1)] * 2, dimension_numbers=('NCHW', 'OIHW', 'NCHW'))
    return out + b[None, :, None, None]


def kernel(template, search, params):
    def backbone(pc, npoints, mlps):
        xyz0 = pc[..., :3]
        c0 = _fps_call(xyz0, npoints[0])
        f0 = _sa_call(c0, xyz0, xyz0, _fold_bn(mlps[0]), 32, 0.3, True)
        c1 = _fps_call(c0, npoints[1])
        f1 = _sa_call(c1, c0, f0, _fold_bn(mlps[1]), 32, 0.5, False)
        c2 = c1[:, :npoints[2]]
        f2 = _sa_call(c2, c1, f1, _fold_bn(mlps[2]), 32, 0.7, False)
        return c2, f2

    nt = template.shape[1]
    ns = search.shape[1]
    mlps = [params['sa0'], params['sa1'], params['sa2']]
    t_xyz, t_feat = backbone(template, [nt // 2, nt // 4, nt // 8], mlps)
    s_xyz, s_feat = backbone(search, [ns // 2, ns // 4, ns // 8], mlps)

    fus = s_feat
    search_xyz = s_xyz

    score_pm = _head_call(fus, params['fc_cla'], sigmoid=True)
    fxf = jnp.concatenate([search_xyz, fus], axis=2)
    off = _head_call(fxf, params['vote'])
    offset = off[:, :, :3]
    fus = fus + off[:, :, 3:]
    temp_sel = search_xyz - offset

    tpool = _sa_call(temp_sel, t_xyz,
                     jnp.concatenate([t_xyz, t_feat], axis=2), [], 8, 1.0, True)
    spool = _sa_call(search_xyz, s_xyz,
                     jnp.concatenate([s_xyz, s_feat], axis=2), [], 8, 1.0, True)

    pf = jnp.concatenate([score_pm, tpool, spool, fus], axis=2)
    po = _head_call(pf, params['proposal'])
    po = jnp.concatenate([po, search_xyz], axis=2)

    vox = _voxelize(po, search_xyz)
    x = jnp.transpose(vox, (0, 1, 4, 3, 2))
    for (w, b) in params['cml']:
        x = jax.nn.relu(_conv3d(x, w, b, (2, 1, 1)))
    bsz, c, d, h, wd = x.shape
    x = x.reshape(bsz, c * d, h, wd)
    hh = jax.nn.relu(_conv2d(x, *params['rpn']['stem']))
    pred_hm = jax.nn.sigmoid(_conv2d(hh, *params['rpn']['hm']))
    pred_loc = _conv2d(hh, *params['rpn']['loc'])
    pred_z = _conv2d(hh, *params['rpn']['z'])
    return pred_hm, pred_loc, pred_z
